# Initial kernel scaffold; baseline (speedup 1.0000x reference)
import jax
import jax.numpy as jnp
from jax import lax
from jax.experimental import pallas as pl
from jax.experimental.pallas import tpu as pltpu
from jax.experimental.pallas import tpu_sc as plsc

N_PTS = 3200000
N_SEG = 100000
NW = 32
PW = 3128
KW = PW * 4
TAB4 = 4 * KW
OUT_PAD = NW * KW
BLKP = 2048
FLAT_BLK = BLKP * 4
NG = FLAT_BLK // 16
NBLK16 = N_PTS // 16


def _worker_body(lidar_hbm, idx_hbm, out_hbm, dbuf, ibuf, table, sbuf):
    iota = lax.iota(jnp.int32, 16)
    iota_div4 = lax.shift_right_logical(iota, 2)
    comp = jnp.bitwise_and(iota, 3)
    class_off = iota_div4 * KW

    wid = lax.axis_index("s") * 2 + lax.axis_index("c")
    t_lo = (wid * PW).astype(jnp.int32)
    t_hi = t_lo + PW

    def searchsorted(t):
        def cond(c):
            return c[0] < c[1]

        def step(c):
            lo, hi = c
            mid = lax.div(lo + hi, jnp.int32(2))
            pltpu.sync_copy(idx_hbm.at[pl.ds(mid * 16, 16)], sbuf)
            first = jnp.min(sbuf[...])
            go_right = first < t
            return (jnp.where(go_right, mid + 1, lo),
                    jnp.where(go_right, hi, mid))

        lo, _ = lax.while_loop(cond, step, (jnp.int32(0), jnp.int32(NBLK16)))
        bprev = jnp.maximum(lo - 1, 0)
        pltpu.sync_copy(idx_hbm.at[pl.ds(bprev * 16, 16)], sbuf)
        cnt = jnp.sum((sbuf[...] < t).astype(jnp.int32))
        return jnp.where(lo > 0, bprev * 16 + cnt, 0)

    s = searchsorted(t_lo)
    e = searchsorted(t_hi)
    s0 = jnp.bitwise_and(s, jnp.int32(-8))
    e0 = jnp.minimum(jnp.bitwise_and(e + 7, jnp.int32(-8)), N_PTS)
    nblk = lax.shift_right_logical(e0 - s0 + (BLKP - 1), 11)

    def init_body(i, _):
        table[pl.ds(i * 16, 16)] = jnp.full((16,), jnp.inf, jnp.float32)
        return 0

    lax.fori_loop(0, TAB4 // 16, init_body, 0)

    def blk_body(b, _):
        start = jnp.minimum(s0 + b * BLKP, N_PTS - BLKP)
        pltpu.sync_copy(idx_hbm.at[pl.ds(start, BLKP)], ibuf)
        pltpu.sync_copy(lidar_hbm.at[pl.ds(start * 4, FLAT_BLK)], dbuf)

        def g_body(g, _):
            pidx = plsc.load_gather(ibuf, [g * 4 + iota_div4])
            valid = jnp.logical_and(pidx >= t_lo, pidx < t_hi)
            local = (pidx - t_lo) * 4 + comp
            slot = jnp.clip(local, 0, KW - 1) + class_off
            v = dbuf[pl.ds(g * 16, 16)]
            cur = plsc.load_gather(table, [slot], mask=valid)
            plsc.store_scatter(table, [slot], jnp.minimum(cur, v), mask=valid)
            return 0

        lax.fori_loop(0, NG, g_body, 0)
        return 0

    lax.fori_loop(0, nblk, blk_body, 0)

    def merge_body(i, _):
        a = jnp.minimum(table[pl.ds(i * 16, 16)], table[pl.ds(KW + i * 16, 16)])
        b = jnp.minimum(table[pl.ds(2 * KW + i * 16, 16)],
                        table[pl.ds(3 * KW + i * 16, 16)])
        table[pl.ds(i * 16, 16)] = jnp.minimum(a, b)
        return 0

    lax.fori_loop(0, KW // 16, merge_body, 0)
    pltpu.sync_copy(table.at[pl.ds(0, KW)], out_hbm.at[pl.ds(wid * KW, KW)])


@jax.jit
def _segment_min_sc(lidar_flat, index):
    mesh = plsc.VectorSubcoreMesh(core_axis_name="c", subcore_axis_name="s")
    run = pl.kernel(
        _worker_body,
        mesh=mesh,
        out_type=jax.ShapeDtypeStruct((OUT_PAD,), jnp.float32),
        scratch_types=[
            pltpu.VMEM((FLAT_BLK,), jnp.float32),
            pltpu.VMEM((BLKP,), jnp.int32),
            pltpu.VMEM((TAB4,), jnp.float32),
            pltpu.VMEM((16,), jnp.int32),
        ],
    )
    return run(lidar_flat, index)


def kernel(lidar, index):
    out_flat = _segment_min_sc(lidar.reshape(-1), index)
    groupby_min = out_flat[: N_SEG * 4].reshape(N_SEG, 4)[:, :3]
    return lidar, groupby_min


# speedup vs baseline: 24.4687x; 24.4687x over previous
import jax
import jax.numpy as jnp
from jax import lax
from jax.experimental import pallas as pl
from jax.experimental.pallas import tpu as pltpu
from jax.experimental.pallas import tpu_sc as plsc

N_PTS = 3200000
N_SEG = 100000
N_TILES = N_PTS // 128
NW = 32
PW = 3128
KW = PW * 4
TAB4 = 4 * KW
OUT_PAD = NW * KW
BLKP = 2048
BLKT = BLKP // 128
NG = BLKP // 4
NBLK16 = N_PTS // 16


def _worker_body(lidar_hbm, idx_hbm, out_hbm, dbuf, ibuf, table, sbuf, sem):
    def _copy(src_ref, dst_ref):
        pltpu.async_copy(src_ref, dst_ref, sem).wait()

    iota = lax.iota(jnp.int32, 16)
    iota_div4 = lax.shift_right_logical(iota, 2)
    comp = jnp.bitwise_and(iota, 3)
    class_off = iota_div4 * KW

    wid = lax.axis_index("s") * 2 + lax.axis_index("c")
    t_lo = (wid * PW).astype(jnp.int32)
    t_hi = t_lo + PW

    def searchsorted(t):
        def step(k, base):
            stp = lax.shift_right_logical(jnp.int32(1 << 17), k)
            cand = jnp.minimum(base + stp, jnp.int32(NBLK16))
            off = pl.multiple_of((cand - 1) * 16, 16)
            _copy(idx_hbm.at[pl.ds(off, 16)], sbuf)
            first = sbuf[...][0]
            take = jnp.logical_and(base + stp <= NBLK16, first < t)
            return jnp.where(take, cand, base)

        return lax.fori_loop(0, 18, step, jnp.int32(0))

    s0 = jnp.bitwise_and(jnp.maximum(searchsorted(t_lo) - 1, 0) * 16,
                         jnp.int32(-128))
    e0 = jnp.minimum(jnp.bitwise_and(searchsorted(t_hi) * 16 + 127,
                                     jnp.int32(-128)), N_PTS)
    nblk = lax.shift_right_logical(e0 - s0 + (BLKP - 1), 11)

    def init_body(i, _):
        table[pl.ds(i * 16, 16)] = jnp.full((16,), jnp.inf, jnp.float32)
        return 0

    lax.fori_loop(0, TAB4 // 16, init_body, 0)

    def blk_body(b, _):
        start = pl.multiple_of(jnp.minimum(s0 + b * BLKP, N_PTS - BLKP), 128)
        _copy(idx_hbm.at[pl.ds(start, BLKP)], ibuf)
        _copy(lidar_hbm.at[pl.ds(lax.shift_right_logical(start, 7), BLKT)],
              dbuf)

        def g_body(g, _):
            pidx = plsc.load_gather(ibuf, [g * 4 + iota_div4])
            valid = jnp.logical_and(pidx >= t_lo, pidx < t_hi)
            local = (pidx - t_lo) * 4 + comp
            slot = jnp.clip(local, 0, KW - 1) + class_off
            tvec = jnp.broadcast_to(lax.shift_right_logical(g, 5), (16,))
            qvec = jnp.bitwise_and(g, 31) * 4 + iota_div4
            v = plsc.load_gather(dbuf, [tvec, comp, qvec])
            cur = plsc.load_gather(table, [slot], mask=valid)
            plsc.store_scatter(table, [slot], jnp.minimum(cur, v), mask=valid)
            return 0

        lax.fori_loop(0, NG, g_body, 0)
        return 0

    lax.fori_loop(0, nblk, blk_body, 0)

    def merge_body(i, _):
        a = jnp.minimum(table[pl.ds(i * 16, 16)], table[pl.ds(KW + i * 16, 16)])
        b = jnp.minimum(table[pl.ds(2 * KW + i * 16, 16)],
                        table[pl.ds(3 * KW + i * 16, 16)])
        table[pl.ds(i * 16, 16)] = jnp.minimum(a, b)
        return 0

    lax.fori_loop(0, KW // 16, merge_body, 0)
    _copy(table.at[pl.ds(0, KW)],
          out_hbm.at[pl.ds(pl.multiple_of(wid * KW, 16), KW)])


@jax.jit
def _segment_min_sc(lidar_t, index):
    mesh = plsc.VectorSubcoreMesh(core_axis_name="c", subcore_axis_name="s")
    run = pl.kernel(
        _worker_body,
        mesh=mesh,
        compiler_params=pltpu.CompilerParams(needs_layout_passes=False),
        out_type=jax.ShapeDtypeStruct((OUT_PAD,), jnp.float32),
        scratch_types=[
            pltpu.VMEM((BLKT, 4, 128), jnp.float32),
            pltpu.VMEM((BLKP,), jnp.int32),
            pltpu.VMEM((TAB4,), jnp.float32),
            pltpu.VMEM((16,), jnp.int32),
            pltpu.SemaphoreType.DMA,
        ],
    )
    return run(lidar_t, index)


def kernel(lidar, index):
    lidar_t = lidar.reshape(N_TILES, 128, 4).transpose(0, 2, 1)
    out_flat = _segment_min_sc(lidar_t, index)
    groupby_min = out_flat[: N_SEG * 4].reshape(N_SEG, 4)[:, :3]
    return lidar, groupby_min
